# baseline (speedup 1.0000x reference)
import jax
import jax.numpy as jnp
from jax import lax
from jax.experimental import pallas as pl
from jax.experimental.pallas import tpu as pltpu
from jax.experimental.pallas import tpu_sc as plsc

B, N, F_IN, H = 64, 512, 512, 768
ROWS = B * N
HP = H // 2
NSPLIT = 2
RS = ROWS // NSPLIT

_NC, _NS = 2, 16
_NW = _NC * _NS
_RPW = RS // _NW
_CHUNK = 64
_NCHUNK = _RPW // _CHUNK
_NBUF = 2


def _sc_gather_body(in_table, out_table, din_hbm, dout_hbm,
                    gin_hbm, gout_hbm,
                    idx_a, idx_b, bufs_a, bufs_b, gsems_a, gsems_b,
                    wsems_a, wsems_b):
    wid = lax.axis_index("s") * _NC + lax.axis_index("c")
    base = wid * _RPW

    pltpu.sync_copy(din_hbm.at[pl.ds(base, _RPW)], idx_a)
    pltpu.sync_copy(dout_hbm.at[pl.ds(base, _RPW)], idx_b)

    def start_gather(c, b):
        s = pl.ds(c * _CHUNK, _CHUNK)
        pltpu.async_copy(in_table.at[idx_a.at[s]], bufs_a.at[b], gsems_a[b])
        pltpu.async_copy(out_table.at[idx_b.at[s]], bufs_b.at[b], gsems_b[b])

    def wait_gather(b):
        pltpu.make_async_copy(in_table.at[idx_a.at[pl.ds(0, _CHUNK)]],
                              bufs_a.at[b], gsems_a[b]).wait()
        pltpu.make_async_copy(out_table.at[idx_b.at[pl.ds(0, _CHUNK)]],
                              bufs_b.at[b], gsems_b[b]).wait()

    def start_write(c, b):
        off = base + c * _CHUNK
        pltpu.async_copy(bufs_a.at[b], gin_hbm.at[pl.ds(off, _CHUNK)],
                         wsems_a[b])
        pltpu.async_copy(bufs_b.at[b], gout_hbm.at[pl.ds(off, _CHUNK)],
                         wsems_b[b])

    def wait_write(b):
        pltpu.make_async_copy(bufs_a.at[b], gin_hbm.at[pl.ds(0, _CHUNK)],
                              wsems_a[b]).wait()
        pltpu.make_async_copy(bufs_b.at[b], gout_hbm.at[pl.ds(0, _CHUNK)],
                              wsems_b[b]).wait()

    for b in range(_NBUF):
        start_gather(b, b)

    def pair(g, _):
        for b in range(_NBUF):
            c = _NBUF * g + b
            wait_gather(b)
            start_write(c, b)
        for b in range(_NBUF):
            c = _NBUF * g + b
            wait_write(b)

            @pl.when(c + _NBUF < _NCHUNK)
            def _():
                start_gather(c + _NBUF, b)
        return ()

    lax.fori_loop(0, _NCHUNK // _NBUF, pair, (), unroll=False)


_sc_gather = pl.kernel(
    _sc_gather_body,
    out_type=(
        jax.ShapeDtypeStruct((RS, HP), jnp.int32),
        jax.ShapeDtypeStruct((RS, HP), jnp.int32),
    ),
    mesh=plsc.VectorSubcoreMesh(core_axis_name="c", subcore_axis_name="s"),
    scratch_types=[
        pltpu.VMEM((_RPW,), jnp.int32),
        pltpu.VMEM((_RPW,), jnp.int32),
        pltpu.VMEM((_NBUF, _CHUNK, HP), jnp.int32),
        pltpu.VMEM((_NBUF, _CHUNK, HP), jnp.int32),
        [pltpu.SemaphoreType.DMA] * _NBUF,
        [pltpu.SemaphoreType.DMA] * _NBUF,
        [pltpu.SemaphoreType.DMA] * _NBUF,
        [pltpu.SemaphoreType.DMA] * _NBUF,
    ],
)


def _unpack_lo_hi(g):
    lo = lax.bitcast_convert_type(g << 16, jnp.float32)
    hi = lax.bitcast_convert_type(g & jnp.int32(-65536), jnp.float32)
    return lo, hi


def _mm_body(x_ref, w_ref, b_ref, gin_ref, gout_ref, prev_ref, o_ref):
    del prev_ref
    acc = jnp.dot(x_ref[...].astype(jnp.bfloat16),
                  w_ref[...].astype(jnp.bfloat16),
                  preferred_element_type=jnp.float32)
    acc = acc + b_ref[...]
    lo_i, hi_i = _unpack_lo_hi(gin_ref[...])
    lo_o, hi_o = _unpack_lo_hi(gout_ref[...])
    o_ref[:, :HP] = acc[:, :HP] + lo_i + lo_o
    o_ref[:, HP:] = acc[:, HP:] + hi_i + hi_o


_BM = 512


def _tc_matmul(x_half, w, b, gin, gout, prev, split):
    grid = (RS // _BM,)
    blk0 = RS // _BM * split
    return pl.pallas_call(
        _mm_body,
        grid=grid,
        in_specs=[
            pl.BlockSpec((_BM, F_IN), lambda i: (i, 0)),
            pl.BlockSpec((F_IN, H), lambda i: (0, 0)),
            pl.BlockSpec((1, H), lambda i: (0, 0)),
            pl.BlockSpec((_BM, HP), lambda i: (i, 0)),
            pl.BlockSpec((_BM, HP), lambda i: (i, 0)),
            pl.BlockSpec(memory_space=pl.ANY),
        ],
        out_specs=pl.BlockSpec((_BM, H), lambda i: (i + blk0, 0)),
        out_shape=jax.ShapeDtypeStruct((ROWS, H), jnp.float32),
        input_output_aliases={5: 0},
    )(x_half, w, b, gin, gout, prev)


def _pack_table(t):
    u = lax.bitcast_convert_type(t.astype(jnp.bfloat16), jnp.uint16)
    u = u.astype(jnp.uint32)
    packed = u[:, :HP] | (u[:, HP:] << 16)
    return lax.bitcast_convert_type(packed, jnp.int32)


def kernel(x, in_degree, out_degree, W_node, b_node, in_table, out_table):
    x2 = x.reshape(ROWS, F_IN)
    din = in_degree.reshape(ROWS).astype(jnp.int32)
    dout = out_degree.reshape(ROWS).astype(jnp.int32)
    tin = _pack_table(in_table)
    tout = _pack_table(out_table)
    b2 = b_node.reshape(1, H)

    gs = [_sc_gather(tin, tout, din[s * RS:(s + 1) * RS],
                     dout[s * RS:(s + 1) * RS]) for s in range(NSPLIT)]

    out = jnp.zeros((ROWS, H), jnp.float32)
    for s in range(NSPLIT):
        gin, gout = gs[s]
        out = _tc_matmul(x2[s * RS:(s + 1) * RS], W_node, b2, gin, gout,
                         out, s)
    return out.reshape(B, N, H)


# speedup vs baseline: 1.1503x; 1.1503x over previous
import jax
import jax.numpy as jnp
from jax import lax
from jax.experimental import pallas as pl
from jax.experimental.pallas import tpu as pltpu
from jax.experimental.pallas import tpu_sc as plsc

B, N, F_IN, H = 64, 512, 512, 768
ROWS = B * N
HP = H // 2
NSPLIT = 2
RS = ROWS // NSPLIT

_NC, _NS = 2, 16
_NW = _NC * _NS
_RPW = RS // _NW
_CHUNK = 64
_NCHUNK = _RPW // _CHUNK
_NBUF = 2


def _sc_gather_body(in_table, out_table, din_hbm, dout_hbm,
                    gin_hbm, gout_hbm,
                    idx_a, idx_b, bufs_a, bufs_b, gsems_a, gsems_b,
                    wsems_a, wsems_b):
    wid = lax.axis_index("s") * _NC + lax.axis_index("c")
    base = wid * _RPW

    pltpu.sync_copy(din_hbm.at[pl.ds(base, _RPW)], idx_a)
    pltpu.sync_copy(dout_hbm.at[pl.ds(base, _RPW)], idx_b)

    def start_gather(c, b):
        s = pl.ds(c * _CHUNK, _CHUNK)
        pltpu.async_copy(in_table.at[idx_a.at[s]], bufs_a.at[b], gsems_a[b])
        pltpu.async_copy(out_table.at[idx_b.at[s]], bufs_b.at[b], gsems_b[b])

    def wait_gather(b):
        pltpu.make_async_copy(in_table.at[idx_a.at[pl.ds(0, _CHUNK)]],
                              bufs_a.at[b], gsems_a[b]).wait()
        pltpu.make_async_copy(out_table.at[idx_b.at[pl.ds(0, _CHUNK)]],
                              bufs_b.at[b], gsems_b[b]).wait()

    def start_write(c, b):
        off = base + c * _CHUNK
        pltpu.async_copy(bufs_a.at[b], gin_hbm.at[pl.ds(off, _CHUNK)],
                         wsems_a[b])
        pltpu.async_copy(bufs_b.at[b], gout_hbm.at[pl.ds(off, _CHUNK)],
                         wsems_b[b])

    def wait_write(b):
        pltpu.make_async_copy(bufs_a.at[b], gin_hbm.at[pl.ds(0, _CHUNK)],
                              wsems_a[b]).wait()
        pltpu.make_async_copy(bufs_b.at[b], gout_hbm.at[pl.ds(0, _CHUNK)],
                              wsems_b[b]).wait()

    for b in range(_NBUF):
        start_gather(b, b)

    def pair(g, _):
        for b in range(_NBUF):
            c = _NBUF * g + b
            wait_gather(b)
            start_write(c, b)
        for b in range(_NBUF):
            c = _NBUF * g + b
            wait_write(b)

            @pl.when(c + _NBUF < _NCHUNK)
            def _():
                start_gather(c + _NBUF, b)
        return ()

    lax.fori_loop(0, _NCHUNK // _NBUF, pair, (), unroll=False)


_sc_gather = pl.kernel(
    _sc_gather_body,
    out_type=(
        jax.ShapeDtypeStruct((RS, HP), jnp.int32),
        jax.ShapeDtypeStruct((RS, HP), jnp.int32),
    ),
    mesh=plsc.VectorSubcoreMesh(core_axis_name="c", subcore_axis_name="s"),
    scratch_types=[
        pltpu.VMEM((_RPW,), jnp.int32),
        pltpu.VMEM((_RPW,), jnp.int32),
        pltpu.VMEM((_NBUF, _CHUNK, HP), jnp.int32),
        pltpu.VMEM((_NBUF, _CHUNK, HP), jnp.int32),
        [pltpu.SemaphoreType.DMA] * _NBUF,
        [pltpu.SemaphoreType.DMA] * _NBUF,
        [pltpu.SemaphoreType.DMA] * _NBUF,
        [pltpu.SemaphoreType.DMA] * _NBUF,
    ],
)


def _unpack_lo_hi(g):
    lo = lax.bitcast_convert_type(g << 16, jnp.float32)
    hi = lax.bitcast_convert_type(g & jnp.int32(-65536), jnp.float32)
    return lo, hi


def _mm_body_first(x_ref, w_ref, b_ref, gin_ref, gout_ref, o_ref):
    _mm_body(x_ref, w_ref, b_ref, gin_ref, gout_ref, None, o_ref)


def _mm_body(x_ref, w_ref, b_ref, gin_ref, gout_ref, prev_ref, o_ref):
    del prev_ref
    acc = jnp.dot(x_ref[...].astype(jnp.bfloat16),
                  w_ref[...].astype(jnp.bfloat16),
                  preferred_element_type=jnp.float32)
    acc = acc + b_ref[...]
    lo_i, hi_i = _unpack_lo_hi(gin_ref[...])
    lo_o, hi_o = _unpack_lo_hi(gout_ref[...])
    o_ref[:, :HP] = acc[:, :HP] + lo_i + lo_o
    o_ref[:, HP:] = acc[:, HP:] + hi_i + hi_o


_BM = 512


def _tc_matmul(x_half, w, b, gin, gout, prev, split):
    grid = (RS // _BM,)
    blk0 = RS // _BM * split
    in_specs = [
        pl.BlockSpec((_BM, F_IN), lambda i: (i, 0)),
        pl.BlockSpec((F_IN, H), lambda i: (0, 0)),
        pl.BlockSpec((1, H), lambda i: (0, 0)),
        pl.BlockSpec((_BM, HP), lambda i: (i, 0)),
        pl.BlockSpec((_BM, HP), lambda i: (i, 0)),
    ]
    args = [x_half, w, b, gin, gout]
    aliases = {}
    body = _mm_body_first
    if prev is not None:
        in_specs.append(pl.BlockSpec(memory_space=pl.ANY))
        args.append(prev)
        aliases = {5: 0}
        body = _mm_body
    return pl.pallas_call(
        body,
        grid=grid,
        in_specs=in_specs,
        out_specs=pl.BlockSpec((_BM, H), lambda i: (i + blk0, 0)),
        out_shape=jax.ShapeDtypeStruct((ROWS, H), jnp.float32),
        input_output_aliases=aliases,
    )(*args)


def _pack_table(t):
    u = lax.bitcast_convert_type(t.astype(jnp.bfloat16), jnp.uint16)
    u = u.astype(jnp.uint32)
    packed = u[:, :HP] | (u[:, HP:] << 16)
    return lax.bitcast_convert_type(packed, jnp.int32)


def kernel(x, in_degree, out_degree, W_node, b_node, in_table, out_table):
    x2 = x.reshape(ROWS, F_IN)
    din = in_degree.reshape(ROWS).astype(jnp.int32)
    dout = out_degree.reshape(ROWS).astype(jnp.int32)
    tin = _pack_table(in_table)
    tout = _pack_table(out_table)
    b2 = b_node.reshape(1, H)

    gs = [_sc_gather(tin, tout, din[s * RS:(s + 1) * RS],
                     dout[s * RS:(s + 1) * RS]) for s in range(NSPLIT)]

    out = None
    for s in range(NSPLIT):
        gin, gout = gs[s]
        out = _tc_matmul(x2[s * RS:(s + 1) * RS], W_node, b2, gin, gout,
                         out, s)
    return out.reshape(B, N, H)


# speedup vs baseline: 1.3047x; 1.1342x over previous
import jax
import jax.numpy as jnp
from jax import lax
from jax.experimental import pallas as pl
from jax.experimental.pallas import tpu as pltpu
from jax.experimental.pallas import tpu_sc as plsc

B, N, F_IN, H = 64, 512, 512, 768
ROWS = B * N
HP = H // 2
NSPLIT = 1
RS = ROWS // NSPLIT

_NC, _NS = 2, 16
_NW = _NC * _NS
_RPW = RS // _NW
_CHUNK = 64
_NCHUNK = _RPW // _CHUNK
_NBUF = 2


def _sc_gather_body(in_table, out_table, din_hbm, dout_hbm,
                    gin_hbm, gout_hbm,
                    idx_a, idx_b, bufs_a, bufs_b, gsems_a, gsems_b,
                    wsems_a, wsems_b):
    wid = lax.axis_index("s") * _NC + lax.axis_index("c")
    base = wid * _RPW

    pltpu.sync_copy(din_hbm.at[pl.ds(base, _RPW)], idx_a)
    pltpu.sync_copy(dout_hbm.at[pl.ds(base, _RPW)], idx_b)

    def start_gather(c, b):
        s = pl.ds(c * _CHUNK, _CHUNK)
        pltpu.async_copy(in_table.at[idx_a.at[s]], bufs_a.at[b], gsems_a[b])
        pltpu.async_copy(out_table.at[idx_b.at[s]], bufs_b.at[b], gsems_b[b])

    def wait_gather(b):
        pltpu.make_async_copy(in_table.at[idx_a.at[pl.ds(0, _CHUNK)]],
                              bufs_a.at[b], gsems_a[b]).wait()
        pltpu.make_async_copy(out_table.at[idx_b.at[pl.ds(0, _CHUNK)]],
                              bufs_b.at[b], gsems_b[b]).wait()

    def start_write(c, b):
        off = base + c * _CHUNK
        pltpu.async_copy(bufs_a.at[b], gin_hbm.at[pl.ds(off, _CHUNK)],
                         wsems_a[b])
        pltpu.async_copy(bufs_b.at[b], gout_hbm.at[pl.ds(off, _CHUNK)],
                         wsems_b[b])

    def wait_write(b):
        pltpu.make_async_copy(bufs_a.at[b], gin_hbm.at[pl.ds(0, _CHUNK)],
                              wsems_a[b]).wait()
        pltpu.make_async_copy(bufs_b.at[b], gout_hbm.at[pl.ds(0, _CHUNK)],
                              wsems_b[b]).wait()

    for b in range(_NBUF):
        start_gather(b, b)

    def pair(g, _):
        for b in range(_NBUF):
            c = _NBUF * g + b
            wait_gather(b)
            start_write(c, b)
        for b in range(_NBUF):
            c = _NBUF * g + b
            wait_write(b)

            @pl.when(c + _NBUF < _NCHUNK)
            def _():
                start_gather(c + _NBUF, b)
        return ()

    lax.fori_loop(0, _NCHUNK // _NBUF, pair, (), unroll=False)


_sc_gather = pl.kernel(
    _sc_gather_body,
    out_type=(
        jax.ShapeDtypeStruct((RS, HP), jnp.int32),
        jax.ShapeDtypeStruct((RS, HP), jnp.int32),
    ),
    mesh=plsc.VectorSubcoreMesh(core_axis_name="c", subcore_axis_name="s"),
    scratch_types=[
        pltpu.VMEM((_RPW,), jnp.int32),
        pltpu.VMEM((_RPW,), jnp.int32),
        pltpu.VMEM((_NBUF, _CHUNK, HP), jnp.int32),
        pltpu.VMEM((_NBUF, _CHUNK, HP), jnp.int32),
        [pltpu.SemaphoreType.DMA] * _NBUF,
        [pltpu.SemaphoreType.DMA] * _NBUF,
        [pltpu.SemaphoreType.DMA] * _NBUF,
        [pltpu.SemaphoreType.DMA] * _NBUF,
    ],
)


def _unpack_lo_hi(g):
    lo = lax.bitcast_convert_type(g << 16, jnp.float32)
    hi = lax.bitcast_convert_type(g & jnp.int32(-65536), jnp.float32)
    return lo, hi


def _mm_body_first(x_ref, w_ref, b_ref, gin_ref, gout_ref, o_ref):
    _mm_body(x_ref, w_ref, b_ref, gin_ref, gout_ref, None, o_ref)


def _mm_body(x_ref, w_ref, b_ref, gin_ref, gout_ref, prev_ref, o_ref):
    del prev_ref
    acc = jnp.dot(x_ref[...].astype(jnp.bfloat16),
                  w_ref[...].astype(jnp.bfloat16),
                  preferred_element_type=jnp.float32)
    acc = acc + b_ref[...]
    lo_i, hi_i = _unpack_lo_hi(gin_ref[...])
    lo_o, hi_o = _unpack_lo_hi(gout_ref[...])
    o_ref[:, :HP] = acc[:, :HP] + lo_i + lo_o
    o_ref[:, HP:] = acc[:, HP:] + hi_i + hi_o


_BM = 512


def _tc_matmul(x_half, w, b, gin, gout, prev, split):
    grid = (RS // _BM,)
    blk0 = RS // _BM * split
    in_specs = [
        pl.BlockSpec((_BM, F_IN), lambda i: (i, 0)),
        pl.BlockSpec((F_IN, H), lambda i: (0, 0)),
        pl.BlockSpec((1, H), lambda i: (0, 0)),
        pl.BlockSpec((_BM, HP), lambda i: (i, 0)),
        pl.BlockSpec((_BM, HP), lambda i: (i, 0)),
    ]
    args = [x_half, w, b, gin, gout]
    aliases = {}
    body = _mm_body_first
    if prev is not None:
        in_specs.append(pl.BlockSpec(memory_space=pl.ANY))
        args.append(prev)
        aliases = {5: 0}
        body = _mm_body
    return pl.pallas_call(
        body,
        grid=grid,
        in_specs=in_specs,
        out_specs=pl.BlockSpec((_BM, H), lambda i: (i + blk0, 0)),
        out_shape=jax.ShapeDtypeStruct((ROWS, H), jnp.float32),
        input_output_aliases=aliases,
    )(*args)


def _pack_table(t):
    u = lax.bitcast_convert_type(t.astype(jnp.bfloat16), jnp.uint16)
    u = u.astype(jnp.uint32)
    packed = u[:, :HP] | (u[:, HP:] << 16)
    return lax.bitcast_convert_type(packed, jnp.int32)


def kernel(x, in_degree, out_degree, W_node, b_node, in_table, out_table):
    x2 = x.reshape(ROWS, F_IN)
    din = in_degree.reshape(ROWS).astype(jnp.int32)
    dout = out_degree.reshape(ROWS).astype(jnp.int32)
    tin = _pack_table(in_table)
    tout = _pack_table(out_table)
    b2 = b_node.reshape(1, H)

    gs = [_sc_gather(tin, tout, din[s * RS:(s + 1) * RS],
                     dout[s * RS:(s + 1) * RS]) for s in range(NSPLIT)]

    out = None
    for s in range(NSPLIT):
        gin, gout = gs[s]
        out = _tc_matmul(x2[s * RS:(s + 1) * RS], W_node, b2, gin, gout,
                         out, s)
    return out.reshape(B, N, H)


# speedup vs baseline: 1.4161x; 1.0853x over previous
import jax
import jax.numpy as jnp
from jax import lax
from jax.experimental import pallas as pl
from jax.experimental.pallas import tpu as pltpu
from jax.experimental.pallas import tpu_sc as plsc

B, N, F_IN, H = 64, 512, 512, 768
ROWS = B * N
HP = H // 2
NSPLIT = 1
RS = ROWS // NSPLIT

_NC, _NS = 2, 16
_NW = _NC * _NS
_RPW = RS // _NW
_CHUNK = 64
_NCHUNK = _RPW // _CHUNK
_NBUF = 2


def _sc_gather_body(in_table, out_table, din_hbm, dout_hbm,
                    gin_hbm, gout_hbm,
                    idx_a, idx_b, bufs_a, bufs_b, gsems_a, gsems_b,
                    wsems_a, wsems_b):
    wid = lax.axis_index("s") * _NC + lax.axis_index("c")
    base = wid * _RPW

    pltpu.sync_copy(din_hbm.at[pl.ds(base, _RPW)], idx_a)
    pltpu.sync_copy(dout_hbm.at[pl.ds(base, _RPW)], idx_b)

    def start_gather(c, b):
        s = pl.ds(c * _CHUNK, _CHUNK)
        pltpu.async_copy(in_table.at[idx_a.at[s]], bufs_a.at[b], gsems_a[b])
        pltpu.async_copy(out_table.at[idx_b.at[s]], bufs_b.at[b], gsems_b[b])

    def wait_gather(b):
        pltpu.make_async_copy(in_table.at[idx_a.at[pl.ds(0, _CHUNK)]],
                              bufs_a.at[b], gsems_a[b]).wait()
        pltpu.make_async_copy(out_table.at[idx_b.at[pl.ds(0, _CHUNK)]],
                              bufs_b.at[b], gsems_b[b]).wait()

    def start_write(c, b):
        off = base + c * _CHUNK
        pltpu.async_copy(bufs_a.at[b], gin_hbm.at[pl.ds(off, _CHUNK)],
                         wsems_a[b])
        pltpu.async_copy(bufs_b.at[b], gout_hbm.at[pl.ds(off, _CHUNK)],
                         wsems_b[b])

    def wait_write(b):
        pltpu.make_async_copy(bufs_a.at[b], gin_hbm.at[pl.ds(0, _CHUNK)],
                              wsems_a[b]).wait()
        pltpu.make_async_copy(bufs_b.at[b], gout_hbm.at[pl.ds(0, _CHUNK)],
                              wsems_b[b]).wait()

    for b in range(_NBUF):
        start_gather(b, b)

    def pair(g, _):
        for b in range(_NBUF):
            c = _NBUF * g + b
            wait_gather(b)
            start_write(c, b)
        for b in range(_NBUF):
            c = _NBUF * g + b
            wait_write(b)

            @pl.when(c + _NBUF < _NCHUNK)
            def _():
                start_gather(c + _NBUF, b)
        return ()

    lax.fori_loop(0, _NCHUNK // _NBUF, pair, (), unroll=False)


_sc_gather = pl.kernel(
    _sc_gather_body,
    out_type=(
        jax.ShapeDtypeStruct((RS, HP), jnp.int32),
        jax.ShapeDtypeStruct((RS, HP), jnp.int32),
    ),
    mesh=plsc.VectorSubcoreMesh(core_axis_name="c", subcore_axis_name="s"),
    scratch_types=[
        pltpu.VMEM((_RPW,), jnp.int32),
        pltpu.VMEM((_RPW,), jnp.int32),
        pltpu.VMEM((_NBUF, _CHUNK, HP), jnp.int32),
        pltpu.VMEM((_NBUF, _CHUNK, HP), jnp.int32),
        [pltpu.SemaphoreType.DMA] * _NBUF,
        [pltpu.SemaphoreType.DMA] * _NBUF,
        [pltpu.SemaphoreType.DMA] * _NBUF,
        [pltpu.SemaphoreType.DMA] * _NBUF,
    ],
)


def _unpack_lo_hi(g):
    lo = lax.bitcast_convert_type(g << 16, jnp.float32)
    hi = lax.bitcast_convert_type(g & jnp.int32(-65536), jnp.float32)
    return lo, hi


def _mm_body_first(x_ref, w_ref, b_ref, gin_ref, gout_ref, o_ref):
    _mm_body(x_ref, w_ref, b_ref, gin_ref, gout_ref, None, o_ref)


def _mm_body(x_ref, w_ref, b_ref, gin_ref, gout_ref, prev_ref, o_ref):
    del prev_ref
    acc = jnp.dot(x_ref[...].astype(jnp.bfloat16),
                  w_ref[...].astype(jnp.bfloat16),
                  preferred_element_type=jnp.float32)
    acc = acc + b_ref[...]
    lo_i, hi_i = _unpack_lo_hi(gin_ref[...])
    lo_o, hi_o = _unpack_lo_hi(gout_ref[...])
    o_ref[:, :HP] = acc[:, :HP] + lo_i + lo_o
    o_ref[:, HP:] = acc[:, HP:] + hi_i + hi_o


_BM = 1024


def _tc_matmul(x_half, w, b, gin, gout, prev, split):
    grid = (RS // _BM,)
    blk0 = RS // _BM * split
    in_specs = [
        pl.BlockSpec((_BM, F_IN), lambda i: (i, 0)),
        pl.BlockSpec((F_IN, H), lambda i: (0, 0)),
        pl.BlockSpec((1, H), lambda i: (0, 0)),
        pl.BlockSpec((_BM, HP), lambda i: (i, 0)),
        pl.BlockSpec((_BM, HP), lambda i: (i, 0)),
    ]
    args = [x_half, w, b, gin, gout]
    aliases = {}
    body = _mm_body_first
    if prev is not None:
        in_specs.append(pl.BlockSpec(memory_space=pl.ANY))
        args.append(prev)
        aliases = {5: 0}
        body = _mm_body
    return pl.pallas_call(
        body,
        grid=grid,
        in_specs=in_specs,
        out_specs=pl.BlockSpec((_BM, H), lambda i: (i + blk0, 0)),
        out_shape=jax.ShapeDtypeStruct((ROWS, H), jnp.float32),
        input_output_aliases=aliases,
    )(*args)


def _pack_table(t):
    u = lax.bitcast_convert_type(t.astype(jnp.bfloat16), jnp.uint16)
    u = u.astype(jnp.uint32)
    packed = u[:, :HP] | (u[:, HP:] << 16)
    return lax.bitcast_convert_type(packed, jnp.int32)


def kernel(x, in_degree, out_degree, W_node, b_node, in_table, out_table):
    x2 = x.reshape(ROWS, F_IN)
    din = in_degree.reshape(ROWS).astype(jnp.int32)
    dout = out_degree.reshape(ROWS).astype(jnp.int32)
    tin = _pack_table(in_table)
    tout = _pack_table(out_table)
    b2 = b_node.reshape(1, H)

    gs = [_sc_gather(tin, tout, din[s * RS:(s + 1) * RS],
                     dout[s * RS:(s + 1) * RS]) for s in range(NSPLIT)]

    out = None
    for s in range(NSPLIT):
        gin, gout = gs[s]
        out = _tc_matmul(x2[s * RS:(s + 1) * RS], W_node, b2, gin, gout,
                         out, s)
    return out.reshape(B, N, H)


# speedup vs baseline: 1.4294x; 1.0094x over previous
import jax
import jax.numpy as jnp
from jax import lax
from jax.experimental import pallas as pl
from jax.experimental.pallas import tpu as pltpu
from jax.experimental.pallas import tpu_sc as plsc

B, N, F_IN, H = 64, 512, 512, 768
ROWS = B * N
HP = H // 2
NSPLIT = 1
RS = ROWS // NSPLIT

_NC, _NS = 2, 16
_NW = _NC * _NS
_RPW = RS // _NW
_CHUNK = 64
_NCHUNK = _RPW // _CHUNK
_NBUF = 2


def _sc_gather_body(in_table, out_table, din_hbm, dout_hbm,
                    gin_hbm, gout_hbm,
                    idx_a, idx_b, bufs_a, bufs_b, gsems_a, gsems_b,
                    wsems_a, wsems_b):
    wid = lax.axis_index("s") * _NC + lax.axis_index("c")
    base = wid * _RPW

    pltpu.sync_copy(din_hbm.at[pl.ds(base, _RPW)], idx_a)
    pltpu.sync_copy(dout_hbm.at[pl.ds(base, _RPW)], idx_b)

    def start_gather(c, b):
        s = pl.ds(c * _CHUNK, _CHUNK)
        pltpu.async_copy(in_table.at[idx_a.at[s]], bufs_a.at[b], gsems_a[b])
        pltpu.async_copy(out_table.at[idx_b.at[s]], bufs_b.at[b], gsems_b[b])

    def wait_gather(b):
        pltpu.make_async_copy(in_table.at[idx_a.at[pl.ds(0, _CHUNK)]],
                              bufs_a.at[b], gsems_a[b]).wait()
        pltpu.make_async_copy(out_table.at[idx_b.at[pl.ds(0, _CHUNK)]],
                              bufs_b.at[b], gsems_b[b]).wait()

    def start_write(c, b):
        off = base + c * _CHUNK
        pltpu.async_copy(bufs_a.at[b], gin_hbm.at[pl.ds(off, _CHUNK)],
                         wsems_a[b])
        pltpu.async_copy(bufs_b.at[b], gout_hbm.at[pl.ds(off, _CHUNK)],
                         wsems_b[b])

    def wait_write(b):
        pltpu.make_async_copy(bufs_a.at[b], gin_hbm.at[pl.ds(0, _CHUNK)],
                              wsems_a[b]).wait()
        pltpu.make_async_copy(bufs_b.at[b], gout_hbm.at[pl.ds(0, _CHUNK)],
                              wsems_b[b]).wait()

    for b in range(_NBUF):
        start_gather(b, b)

    def pair(g, _):
        for b in range(_NBUF):
            c = _NBUF * g + b
            wait_gather(b)
            start_write(c, b)
        for b in range(_NBUF):
            c = _NBUF * g + b
            wait_write(b)

            @pl.when(c + _NBUF < _NCHUNK)
            def _():
                start_gather(c + _NBUF, b)
        return ()

    lax.fori_loop(0, _NCHUNK // _NBUF, pair, (), unroll=False)


_sc_gather = pl.kernel(
    _sc_gather_body,
    out_type=(
        jax.ShapeDtypeStruct((RS, HP), jnp.int32),
        jax.ShapeDtypeStruct((RS, HP), jnp.int32),
    ),
    mesh=plsc.VectorSubcoreMesh(core_axis_name="c", subcore_axis_name="s"),
    scratch_types=[
        pltpu.VMEM((_RPW,), jnp.int32),
        pltpu.VMEM((_RPW,), jnp.int32),
        pltpu.VMEM((_NBUF, _CHUNK, HP), jnp.int32),
        pltpu.VMEM((_NBUF, _CHUNK, HP), jnp.int32),
        [pltpu.SemaphoreType.DMA] * _NBUF,
        [pltpu.SemaphoreType.DMA] * _NBUF,
        [pltpu.SemaphoreType.DMA] * _NBUF,
        [pltpu.SemaphoreType.DMA] * _NBUF,
    ],
)


def _unpack_lo_hi(g):
    lo = lax.bitcast_convert_type(g << 16, jnp.float32)
    hi = lax.bitcast_convert_type(g & jnp.int32(-65536), jnp.float32)
    return lo, hi


def _mm_body_first(x_ref, w_ref, b_ref, gin_ref, gout_ref, o_ref):
    _mm_body(x_ref, w_ref, b_ref, gin_ref, gout_ref, None, o_ref)


def _mm_body(x_ref, w_ref, b_ref, gin_ref, gout_ref, prev_ref, o_ref):
    del prev_ref
    acc = jnp.dot(x_ref[...].astype(jnp.bfloat16),
                  w_ref[...].astype(jnp.bfloat16),
                  preferred_element_type=jnp.float32)
    acc = acc + b_ref[...]
    lo_i, hi_i = _unpack_lo_hi(gin_ref[...])
    lo_o, hi_o = _unpack_lo_hi(gout_ref[...])
    o_ref[:, :HP] = acc[:, :HP] + lo_i + lo_o
    o_ref[:, HP:] = acc[:, HP:] + hi_i + hi_o


_BM = 2048


def _tc_matmul(x_half, w, b, gin, gout, prev, split):
    grid = (RS // _BM,)
    blk0 = RS // _BM * split
    in_specs = [
        pl.BlockSpec((_BM, F_IN), lambda i: (i, 0)),
        pl.BlockSpec((F_IN, H), lambda i: (0, 0)),
        pl.BlockSpec((1, H), lambda i: (0, 0)),
        pl.BlockSpec((_BM, HP), lambda i: (i, 0)),
        pl.BlockSpec((_BM, HP), lambda i: (i, 0)),
    ]
    args = [x_half, w, b, gin, gout]
    aliases = {}
    body = _mm_body_first
    if prev is not None:
        in_specs.append(pl.BlockSpec(memory_space=pl.ANY))
        args.append(prev)
        aliases = {5: 0}
        body = _mm_body
    return pl.pallas_call(
        body,
        grid=grid,
        in_specs=in_specs,
        out_specs=pl.BlockSpec((_BM, H), lambda i: (i + blk0, 0)),
        out_shape=jax.ShapeDtypeStruct((ROWS, H), jnp.float32),
        input_output_aliases=aliases,
    )(*args)


def _pack_table(t):
    u = lax.bitcast_convert_type(t.astype(jnp.bfloat16), jnp.uint16)
    u = u.astype(jnp.uint32)
    packed = u[:, :HP] | (u[:, HP:] << 16)
    return lax.bitcast_convert_type(packed, jnp.int32)


def kernel(x, in_degree, out_degree, W_node, b_node, in_table, out_table):
    x2 = x.reshape(ROWS, F_IN)
    din = in_degree.reshape(ROWS).astype(jnp.int32)
    dout = out_degree.reshape(ROWS).astype(jnp.int32)
    tin = _pack_table(in_table)
    tout = _pack_table(out_table)
    b2 = b_node.reshape(1, H)

    gs = [_sc_gather(tin, tout, din[s * RS:(s + 1) * RS],
                     dout[s * RS:(s + 1) * RS]) for s in range(NSPLIT)]

    out = None
    for s in range(NSPLIT):
        gin, gout = gs[s]
        out = _tc_matmul(x2[s * RS:(s + 1) * RS], W_node, b2, gin, gout,
                         out, s)
    return out.reshape(B, N, H)


# speedup vs baseline: 1.4354x; 1.0042x over previous
import jax
import jax.numpy as jnp
from jax import lax
from jax.experimental import pallas as pl
from jax.experimental.pallas import tpu as pltpu
from jax.experimental.pallas import tpu_sc as plsc

B, N, F_IN, H = 64, 512, 512, 768
ROWS = B * N
HP = H // 2
NSPLIT = 1
RS = ROWS // NSPLIT

_NC, _NS = 2, 16
_NW = _NC * _NS
_RPW = RS // _NW
_CHUNK = 32
_NCHUNK = _RPW // _CHUNK
_NBUF = 4


def _sc_gather_body(in_table, out_table, din_hbm, dout_hbm,
                    gin_hbm, gout_hbm,
                    idx_a, idx_b, bufs_a, bufs_b,
                    gsems_a, gsems_b, wsems_a, wsems_b):
    wid = lax.axis_index("s") * _NC + lax.axis_index("c")
    base = wid * _RPW

    pltpu.sync_copy(din_hbm.at[pl.ds(base, _RPW)], idx_a)
    pltpu.sync_copy(dout_hbm.at[pl.ds(base, _RPW)], idx_b)

    def start_gather(c, b):
        s = pl.ds(c * _CHUNK, _CHUNK)
        pltpu.async_copy(in_table.at[idx_a.at[s]], bufs_a.at[b], gsems_a[b])
        pltpu.async_copy(out_table.at[idx_b.at[s]], bufs_b.at[b], gsems_b[b])

    def wait_gather(b):
        pltpu.make_async_copy(in_table.at[idx_a.at[pl.ds(0, _CHUNK)]],
                              bufs_a.at[b], gsems_a[b]).wait()
        pltpu.make_async_copy(out_table.at[idx_b.at[pl.ds(0, _CHUNK)]],
                              bufs_b.at[b], gsems_b[b]).wait()

    def start_write(c, b):
        off = base + c * _CHUNK
        pltpu.async_copy(bufs_a.at[b], gin_hbm.at[pl.ds(off, _CHUNK)],
                         wsems_a[b])
        pltpu.async_copy(bufs_b.at[b], gout_hbm.at[pl.ds(off, _CHUNK)],
                         wsems_b[b])

    def wait_write(b):
        pltpu.make_async_copy(bufs_a.at[b], gin_hbm.at[pl.ds(0, _CHUNK)],
                              wsems_a[b]).wait()
        pltpu.make_async_copy(bufs_b.at[b], gout_hbm.at[pl.ds(0, _CHUNK)],
                              wsems_b[b]).wait()

    for b in range(_NBUF):
        start_gather(b, b)

    def pair(g, _):
        for b in range(_NBUF):
            c = _NBUF * g + b
            wait_gather(b)
            start_write(c, b)
        for b in range(_NBUF):
            c = _NBUF * g + b
            wait_write(b)

            @pl.when(c + _NBUF < _NCHUNK)
            def _():
                start_gather(c + _NBUF, b)
        return ()

    lax.fori_loop(0, _NCHUNK // _NBUF, pair, (), unroll=False)


_sc_gather = pl.kernel(
    _sc_gather_body,
    out_type=(
        jax.ShapeDtypeStruct((RS, HP), jnp.int32),
        jax.ShapeDtypeStruct((RS, HP), jnp.int32),
    ),
    mesh=plsc.VectorSubcoreMesh(core_axis_name="c", subcore_axis_name="s"),
    scratch_types=[
        pltpu.VMEM((_RPW,), jnp.int32),
        pltpu.VMEM((_RPW,), jnp.int32),
        pltpu.VMEM((_NBUF, _CHUNK, HP), jnp.int32),
        pltpu.VMEM((_NBUF, _CHUNK, HP), jnp.int32),
        [pltpu.SemaphoreType.DMA] * _NBUF,
        [pltpu.SemaphoreType.DMA] * _NBUF,
        [pltpu.SemaphoreType.DMA] * _NBUF,
        [pltpu.SemaphoreType.DMA] * _NBUF,
    ],
)


def _unpack_lo_hi(g):
    lo = lax.bitcast_convert_type(g << 16, jnp.float32)
    hi = lax.bitcast_convert_type(g & jnp.int32(-65536), jnp.float32)
    return lo, hi


def _mm_body_first(x_ref, w_ref, b_ref, gin_ref, gout_ref, o_ref):
    _mm_body(x_ref, w_ref, b_ref, gin_ref, gout_ref, None, o_ref)


def _mm_body(x_ref, w_ref, b_ref, gin_ref, gout_ref, prev_ref, o_ref):
    del prev_ref
    acc = jnp.dot(x_ref[...].astype(jnp.bfloat16),
                  w_ref[...].astype(jnp.bfloat16),
                  preferred_element_type=jnp.float32)
    acc = acc + b_ref[...]
    lo_i, hi_i = _unpack_lo_hi(gin_ref[...])
    lo_o, hi_o = _unpack_lo_hi(gout_ref[...])
    o_ref[:, :HP] = acc[:, :HP] + lo_i + lo_o
    o_ref[:, HP:] = acc[:, HP:] + hi_i + hi_o


_BM = 2048


def _tc_matmul(x_half, w, b, gin, gout, prev, split):
    grid = (RS // _BM,)
    blk0 = RS // _BM * split
    in_specs = [
        pl.BlockSpec((_BM, F_IN), lambda i: (i, 0)),
        pl.BlockSpec((F_IN, H), lambda i: (0, 0)),
        pl.BlockSpec((1, H), lambda i: (0, 0)),
        pl.BlockSpec((_BM, HP), lambda i: (i, 0)),
        pl.BlockSpec((_BM, HP), lambda i: (i, 0)),
    ]
    args = [x_half, w, b, gin, gout]
    aliases = {}
    body = _mm_body_first
    if prev is not None:
        in_specs.append(pl.BlockSpec(memory_space=pl.ANY))
        args.append(prev)
        aliases = {5: 0}
        body = _mm_body
    return pl.pallas_call(
        body,
        grid=grid,
        in_specs=in_specs,
        out_specs=pl.BlockSpec((_BM, H), lambda i: (i + blk0, 0)),
        out_shape=jax.ShapeDtypeStruct((ROWS, H), jnp.float32),
        input_output_aliases=aliases,
    )(*args)


def _pack_table(t):
    u = lax.bitcast_convert_type(t.astype(jnp.bfloat16), jnp.uint16)
    u = u.astype(jnp.uint32)
    packed = u[:, :HP] | (u[:, HP:] << 16)
    return lax.bitcast_convert_type(packed, jnp.int32)


def kernel(x, in_degree, out_degree, W_node, b_node, in_table, out_table):
    x2 = x.reshape(ROWS, F_IN)
    din = in_degree.reshape(ROWS).astype(jnp.int32)
    dout = out_degree.reshape(ROWS).astype(jnp.int32)
    tin = _pack_table(in_table)
    tout = _pack_table(out_table)
    b2 = b_node.reshape(1, H)

    gs = [_sc_gather(tin, tout, din[s * RS:(s + 1) * RS],
                     dout[s * RS:(s + 1) * RS]) for s in range(NSPLIT)]

    out = None
    for s in range(NSPLIT):
        gin, gout = gs[s]
        out = _tc_matmul(x2[s * RS:(s + 1) * RS], W_node, b2, gin, gout,
                         out, s)
    return out.reshape(B, N, H)
